# baseline (speedup 1.0000x reference)
import jax
import jax.numpy as jnp
from jax.experimental import pallas as pl

_B = 4096
_F = 128
_H = 64
_K = 15
_EPS = 1e-08
_TILE = 256
_CW = 32
_NC = (_F * _F) // _CW
_CG = _NC // 128


def _body(x_ref, Ws1_ref, bs1_ref, Ws2_ref, bs2_ref, Wo1_ref, bo1_ref,
          Wo2_ref, bo2_ref, Wt1_ref, bt1_ref, Wt2_ref, bt2_ref,
          ratio_ref, pv_ref, op_ref, mag_ref, ent_ref):
    xb = x_ref[...]

    hs = jnp.maximum(jnp.dot(xb, Ws1_ref[...]) + bs1_ref[...], 0.0)
    logits = jnp.dot(hs, Ws2_ref[...]) + bs2_ref[...]
    sel = jax.nn.sigmoid(logits)

    ent_part = jnp.sum(-sel * jnp.log(sel + 1e-08)).reshape(1, 1)

    T = sel.shape[0]
    s4 = sel.reshape(T, _CW, _CG, 128)
    cmax = jnp.max(s4, axis=1)
    cid = (jax.lax.broadcasted_iota(jnp.int32, cmax.shape, 1) * 128
           + jax.lax.broadcasted_iota(jnp.int32, cmax.shape, 2))
    work = cmax
    chunk_ids = []
    for _ in range(_K):
        m = jnp.max(work, axis=(1, 2), keepdims=True)
        sel_cid = jnp.min(jnp.where(work == m, cid, _NC),
                          axis=(1, 2), keepdims=True)
        work = jnp.where(cid == sel_cid, -1.0, work)
        chunk_ids.append(sel_cid.reshape(T, 1))
    csel = jnp.concatenate(chunk_ids, axis=1)
    cl = jnp.broadcast_to((csel % 128).reshape(T, 1, 1, _K),
                          (T, _CW, _CG, _K))
    gath = jnp.take_along_axis(s4, cl, axis=3)
    cg_s = (csel // 128).reshape(T, 1, 1, _K)
    g_iota = jax.lax.broadcasted_iota(jnp.int32, gath.shape, 2)
    cand = jnp.sum(jnp.where(g_iota == cg_s, gath, 0.0), axis=2)
    cand_idx = ((csel * _CW).reshape(T, 1, _K)
                + jax.lax.broadcasted_iota(jnp.int32, cand.shape, 1))
    vals, idxs = [], []
    for _ in range(_K):
        m = jnp.max(cand, axis=(1, 2), keepdims=True)
        o = jnp.min(jnp.where(cand == m, cand_idx, _F * _F),
                    axis=(1, 2), keepdims=True)
        cand = jnp.where(cand_idx == o, -1.0, cand)
        vals.append(m.reshape(T, 1))
        idxs.append(o.reshape(T, 1))
    top_vals = jnp.concatenate(vals, axis=1)

    ht = jnp.maximum(jnp.dot(xb, Wt1_ref[...]) + bt1_ref[...], 0.0)
    tr = jnp.dot(ht, Wt2_ref[...]) + bt2_ref[...]

    ho = jnp.maximum(jnp.dot(xb, Wo1_ref[...]) + bo1_ref[...], 0.0)
    ol = jnp.dot(ho, Wo2_ref[...]) + bo2_ref[...]
    ol = ol - jax.lax.stop_gradient(jnp.max(ol, axis=1, keepdims=True))
    eol = jnp.exp(ol)
    opw = eol / jnp.sum(eol, axis=1, keepdims=True)

    lane = jax.lax.broadcasted_iota(jnp.int32, tr.shape, 1)
    w0 = opw[:, 0:1]
    w1 = opw[:, 1:2]
    w2 = opw[:, 2:3]
    w3 = opw[:, 3:4]
    cols = []
    for k in range(_K):
        i_k = idxs[k] // _F
        j_k = idxs[k] % _F
        fi = jnp.sum(jnp.where(lane == i_k, tr, 0.0), axis=1, keepdims=True)
        fj = jnp.sum(jnp.where(lane == j_k, tr, 0.0), axis=1, keepdims=True)
        abs_fj = jnp.abs(fj) + _EPS
        ratio = fi / abs_fj
        log_ratio = jnp.log(jnp.abs(fi) + _EPS) - jnp.log(abs_fj)
        combined = (ratio * w0 + log_ratio * w1 + (fi - fj) * w2
                    + (fi * fj) * w3)
        cols.append(combined)
    combined = jnp.concatenate(cols, axis=1)

    ratio_ref[...] = combined

    pv_part = jnp.sum(top_vals, axis=0, keepdims=True)
    mag_part = jnp.sum(jnp.abs(combined), axis=0, keepdims=True)
    op_part = jnp.sum(opw, axis=0, keepdims=True)

    @pl.when(pl.program_id(0) == 0)
    def _init():
        pv_ref[...] = pv_part
        op_ref[...] = op_part
        mag_ref[...] = mag_part
        ent_ref[...] = ent_part

    @pl.when(pl.program_id(0) != 0)
    def _acc():
        pv_ref[...] += pv_part
        op_ref[...] += op_part
        mag_ref[...] += mag_part
        ent_ref[...] += ent_part


def kernel(x, Ws1, bs1, Ws2, bs2, Wo1, bo1, Wo2, bo2, Wt1, bt1, Wt2, bt2):
    grid = (_B // _TILE,)

    def full(a):
        return pl.BlockSpec(a.shape, lambda i: (0,) * a.ndim)

    Ws2 = Ws2.reshape(_H, _NC, _CW).transpose(0, 2, 1).reshape(_H, _F * _F)
    bs2 = bs2.reshape(_NC, _CW).T.reshape(_F * _F)

    bs1r = bs1.reshape(1, _H)
    bs2r = bs2.reshape(1, _F * _F)
    bo1r = bo1.reshape(1, _H)
    bo2r = bo2.reshape(1, 4)
    bt1r = bt1.reshape(1, _H)
    bt2r = bt2.reshape(1, _F)

    out_shapes = (
        jax.ShapeDtypeStruct((_B, _K), jnp.float32),
        jax.ShapeDtypeStruct((1, _K), jnp.float32),
        jax.ShapeDtypeStruct((1, 4), jnp.float32),
        jax.ShapeDtypeStruct((1, _K), jnp.float32),
        jax.ShapeDtypeStruct((1, 1), jnp.float32),
    )
    out_specs = (
        pl.BlockSpec((_TILE, _K), lambda i: (i, 0)),
        pl.BlockSpec((1, _K), lambda i: (0, 0)),
        pl.BlockSpec((1, 4), lambda i: (0, 0)),
        pl.BlockSpec((1, _K), lambda i: (0, 0)),
        pl.BlockSpec((1, 1), lambda i: (0, 0)),
    )
    in_specs = [
        pl.BlockSpec((_TILE, _F), lambda i: (i, 0)),
        full(Ws1), full(bs1r), full(Ws2), full(bs2r),
        full(Wo1), full(bo1r), full(Wo2), full(bo2r),
        full(Wt1), full(bt1r), full(Wt2), full(bt2r),
    ]

    ratio_t, pv, op, mag, ent = pl.pallas_call(
        _body,
        grid=grid,
        in_specs=in_specs,
        out_specs=out_specs,
        out_shape=out_shapes,
    )(x, Ws1, bs1r, Ws2, bs2r, Wo1, bo1r, Wo2, bo2r, Wt1, bt1r, Wt2, bt2r)

    inv_b = 1.0 / _B
    return (ratio_t,
            pv[0] * inv_b,
            op[0] * inv_b,
            mag[0] * inv_b,
            (ent[0, 0] * inv_b).astype(jnp.float32))


# speedup vs baseline: 1.6604x; 1.6604x over previous
import jax
import jax.numpy as jnp
from jax.experimental import pallas as pl

_B = 4096
_F = 128
_H = 64
_K = 15
_EPS = 1e-08
_TILE = 256
_CW = 32
_NC = (_F * _F) // _CW
_CG = _NC // 128


def _body(x_ref, Ws1_ref, bs1_ref, Ws2_ref, bs2_ref, Wo1_ref, bo1_ref,
          Wo2_ref, bo2_ref, Wt1_ref, bt1_ref, Wt2_ref, bt2_ref,
          ratio_ref, pv_ref, op_ref, mag_ref, ent_ref):
    xb = x_ref[...]

    hs = jnp.maximum(jnp.dot(xb, Ws1_ref[...]) + bs1_ref[...], 0.0)
    logits = jnp.dot(hs, Ws2_ref[...]) + bs2_ref[...]
    sel = jax.nn.sigmoid(logits)

    ent_part = jnp.sum(-sel * jnp.log(sel + 1e-08)).reshape(1, 1)

    T = sel.shape[0]
    s4 = sel.reshape(T, _CW, _CG, 128)
    cmax = jnp.max(s4, axis=1)
    cid = (jax.lax.broadcasted_iota(jnp.int32, cmax.shape, 1) * 128
           + jax.lax.broadcasted_iota(jnp.int32, cmax.shape, 2))
    work = cmax
    chunk_ids = []
    for _ in range(_K):
        m = jnp.max(work, axis=(1, 2), keepdims=True)
        sel_cid = jnp.min(jnp.where(work == m, cid, _NC),
                          axis=(1, 2), keepdims=True)
        work = jnp.where(cid == sel_cid, -1.0, work)
        chunk_ids.append(sel_cid.reshape(T, 1))
    csel = jnp.concatenate(chunk_ids, axis=1)
    cl = jnp.broadcast_to((csel % 128).reshape(T, 1, 1, _K),
                          (T, _CW, _CG, _K))
    gath = jnp.take_along_axis(s4, cl, axis=3)
    cg_s = (csel // 128).reshape(T, 1, 1, _K)
    g_iota = jax.lax.broadcasted_iota(jnp.int32, gath.shape, 2)
    cand = jnp.max(jnp.where(g_iota == cg_s, gath, -1.0),
                   axis=2).reshape(T, _K * _CW)
    q_iota = jax.lax.broadcasted_iota(jnp.int32, cand.shape, 1)
    s_pat = q_iota % _K
    w_pat = q_iota // _K
    oidx = jnp.take_along_axis(csel, s_pat, axis=1) * _CW + w_pat
    vals, idxs = [], []
    for _ in range(_K):
        m = jnp.max(cand, axis=1, keepdims=True)
        o = jnp.min(jnp.where(cand == m, oidx, _F * _F),
                    axis=1, keepdims=True)
        cand = jnp.where(oidx == o, -1.0, cand)
        vals.append(m)
        idxs.append(o)
    top_vals = jnp.concatenate(vals, axis=1)

    ht = jnp.maximum(jnp.dot(xb, Wt1_ref[...]) + bt1_ref[...], 0.0)
    tr = jnp.dot(ht, Wt2_ref[...]) + bt2_ref[...]

    ho = jnp.maximum(jnp.dot(xb, Wo1_ref[...]) + bo1_ref[...], 0.0)
    ol = jnp.dot(ho, Wo2_ref[...]) + bo2_ref[...]
    ol = ol - jax.lax.stop_gradient(jnp.max(ol, axis=1, keepdims=True))
    eol = jnp.exp(ol)
    opw = eol / jnp.sum(eol, axis=1, keepdims=True)

    lane = jax.lax.broadcasted_iota(jnp.int32, tr.shape, 1)
    w0 = opw[:, 0:1]
    w1 = opw[:, 1:2]
    w2 = opw[:, 2:3]
    w3 = opw[:, 3:4]
    cols = []
    for k in range(_K):
        i_k = idxs[k] // _F
        j_k = idxs[k] % _F
        fi = jnp.sum(jnp.where(lane == i_k, tr, 0.0), axis=1, keepdims=True)
        fj = jnp.sum(jnp.where(lane == j_k, tr, 0.0), axis=1, keepdims=True)
        abs_fj = jnp.abs(fj) + _EPS
        ratio = fi / abs_fj
        log_ratio = jnp.log(jnp.abs(fi) + _EPS) - jnp.log(abs_fj)
        combined = (ratio * w0 + log_ratio * w1 + (fi - fj) * w2
                    + (fi * fj) * w3)
        cols.append(combined)
    combined = jnp.concatenate(cols, axis=1)

    ratio_ref[...] = combined

    pv_part = jnp.sum(top_vals, axis=0, keepdims=True)
    mag_part = jnp.sum(jnp.abs(combined), axis=0, keepdims=True)
    op_part = jnp.sum(opw, axis=0, keepdims=True)

    @pl.when(pl.program_id(0) == 0)
    def _init():
        pv_ref[...] = pv_part
        op_ref[...] = op_part
        mag_ref[...] = mag_part
        ent_ref[...] = ent_part

    @pl.when(pl.program_id(0) != 0)
    def _acc():
        pv_ref[...] += pv_part
        op_ref[...] += op_part
        mag_ref[...] += mag_part
        ent_ref[...] += ent_part


def kernel(x, Ws1, bs1, Ws2, bs2, Wo1, bo1, Wo2, bo2, Wt1, bt1, Wt2, bt2):
    grid = (_B // _TILE,)

    def full(a):
        return pl.BlockSpec(a.shape, lambda i: (0,) * a.ndim)

    Ws2 = Ws2.reshape(_H, _NC, _CW).transpose(0, 2, 1).reshape(_H, _F * _F)
    bs2 = bs2.reshape(_NC, _CW).T.reshape(_F * _F)

    bs1r = bs1.reshape(1, _H)
    bs2r = bs2.reshape(1, _F * _F)
    bo1r = bo1.reshape(1, _H)
    bo2r = bo2.reshape(1, 4)
    bt1r = bt1.reshape(1, _H)
    bt2r = bt2.reshape(1, _F)

    out_shapes = (
        jax.ShapeDtypeStruct((_B, _K), jnp.float32),
        jax.ShapeDtypeStruct((1, _K), jnp.float32),
        jax.ShapeDtypeStruct((1, 4), jnp.float32),
        jax.ShapeDtypeStruct((1, _K), jnp.float32),
        jax.ShapeDtypeStruct((1, 1), jnp.float32),
    )
    out_specs = (
        pl.BlockSpec((_TILE, _K), lambda i: (i, 0)),
        pl.BlockSpec((1, _K), lambda i: (0, 0)),
        pl.BlockSpec((1, 4), lambda i: (0, 0)),
        pl.BlockSpec((1, _K), lambda i: (0, 0)),
        pl.BlockSpec((1, 1), lambda i: (0, 0)),
    )
    in_specs = [
        pl.BlockSpec((_TILE, _F), lambda i: (i, 0)),
        full(Ws1), full(bs1r), full(Ws2), full(bs2r),
        full(Wo1), full(bo1r), full(Wo2), full(bo2r),
        full(Wt1), full(bt1r), full(Wt2), full(bt2r),
    ]

    ratio_t, pv, op, mag, ent = pl.pallas_call(
        _body,
        grid=grid,
        in_specs=in_specs,
        out_specs=out_specs,
        out_shape=out_shapes,
    )(x, Ws1, bs1r, Ws2, bs2r, Wo1, bo1r, Wo2, bo2r, Wt1, bt1r, Wt2, bt2r)

    inv_b = 1.0 / _B
    return (ratio_t,
            pv[0] * inv_b,
            op[0] * inv_b,
            mag[0] * inv_b,
            (ent[0, 0] * inv_b).astype(jnp.float32))


# speedup vs baseline: 2.0397x; 1.2284x over previous
import jax
import jax.numpy as jnp
from jax.experimental import pallas as pl

_B = 4096
_F = 128
_H = 64
_K = 15
_EPS = 1e-08
_TILE = 256
_CW = 32
_NC = (_F * _F) // _CW
_CG = _NC // 128


def _body(x_ref, Ws1_ref, bs1_ref, Ws2_ref, bs2_ref, Wo1_ref, bo1_ref,
          Wo2_ref, bo2_ref, Wt1_ref, bt1_ref, Wt2_ref, bt2_ref,
          ratio_ref, pv_ref, op_ref, mag_ref, ent_ref):
    xb = x_ref[...]

    hs = jnp.maximum(jnp.dot(xb, Ws1_ref[...]) + bs1_ref[...], 0.0)
    logits = jnp.dot(hs, Ws2_ref[...]) + bs2_ref[...]
    sel = jax.nn.sigmoid(logits)

    ent_part = jnp.sum(-sel * jnp.log(sel + 1e-08)).reshape(1, 1)

    T = sel.shape[0]
    s4 = sel.reshape(T, _CW, _CG, 128)
    cmax = jnp.max(sel.reshape(T, _CW, _NC), axis=1)
    cid = jax.lax.broadcasted_iota(jnp.int32, cmax.shape, 1)
    chunk_ids = []
    for _ in range(_K):
        m = jnp.max(cmax, axis=1, keepdims=True)
        sel_cid = jnp.min(jnp.where(cmax == m, cid, _NC),
                          axis=1, keepdims=True)
        cmax = jnp.where(cid == sel_cid, -1.0, cmax)
        chunk_ids.append(sel_cid)
    csel = jnp.concatenate(chunk_ids, axis=1)
    cl = jnp.broadcast_to((csel % 128).reshape(T, 1, 1, _K),
                          (T, _CW, _CG, _K))
    gath = jnp.take_along_axis(s4, cl, axis=3)
    cg_s = (csel // 128).reshape(T, 1, 1, _K)
    g_iota = jax.lax.broadcasted_iota(jnp.int32, gath.shape, 2)
    cand = jnp.max(jnp.where(g_iota == cg_s, gath, -1.0),
                   axis=2).reshape(T, _K * _CW)
    q_iota = jax.lax.broadcasted_iota(jnp.int32, cand.shape, 1)
    s_pat = q_iota % _K
    w_pat = q_iota // _K
    oidx = jnp.take_along_axis(csel, s_pat, axis=1) * _CW + w_pat
    vals, idxs = [], []
    for _ in range(_K):
        m = jnp.max(cand, axis=1, keepdims=True)
        o = jnp.min(jnp.where(cand == m, oidx, _F * _F),
                    axis=1, keepdims=True)
        cand = jnp.where(oidx == o, -1.0, cand)
        vals.append(m)
        idxs.append(o)
    top_vals = jnp.concatenate(vals, axis=1)

    ht = jnp.maximum(jnp.dot(xb, Wt1_ref[...]) + bt1_ref[...], 0.0)
    tr = jnp.dot(ht, Wt2_ref[...]) + bt2_ref[...]

    ho = jnp.maximum(jnp.dot(xb, Wo1_ref[...]) + bo1_ref[...], 0.0)
    ol = jnp.dot(ho, Wo2_ref[...]) + bo2_ref[...]
    ol = ol - jax.lax.stop_gradient(jnp.max(ol, axis=1, keepdims=True))
    eol = jnp.exp(ol)
    opw = eol / jnp.sum(eol, axis=1, keepdims=True)

    lane = jax.lax.broadcasted_iota(jnp.int32, tr.shape, 1)
    w0 = opw[:, 0:1]
    w1 = opw[:, 1:2]
    w2 = opw[:, 2:3]
    w3 = opw[:, 3:4]
    cols = []
    for k in range(_K):
        i_k = idxs[k] // _F
        j_k = idxs[k] % _F
        fi = jnp.sum(jnp.where(lane == i_k, tr, 0.0), axis=1, keepdims=True)
        fj = jnp.sum(jnp.where(lane == j_k, tr, 0.0), axis=1, keepdims=True)
        abs_fj = jnp.abs(fj) + _EPS
        ratio = fi / abs_fj
        log_ratio = jnp.log(jnp.abs(fi) + _EPS) - jnp.log(abs_fj)
        combined = (ratio * w0 + log_ratio * w1 + (fi - fj) * w2
                    + (fi * fj) * w3)
        cols.append(combined)
    combined = jnp.concatenate(cols, axis=1)

    ratio_ref[...] = combined

    pv_part = jnp.sum(top_vals, axis=0, keepdims=True)
    mag_part = jnp.sum(jnp.abs(combined), axis=0, keepdims=True)
    op_part = jnp.sum(opw, axis=0, keepdims=True)

    @pl.when(pl.program_id(0) == 0)
    def _init():
        pv_ref[...] = pv_part
        op_ref[...] = op_part
        mag_ref[...] = mag_part
        ent_ref[...] = ent_part

    @pl.when(pl.program_id(0) != 0)
    def _acc():
        pv_ref[...] += pv_part
        op_ref[...] += op_part
        mag_ref[...] += mag_part
        ent_ref[...] += ent_part


def kernel(x, Ws1, bs1, Ws2, bs2, Wo1, bo1, Wo2, bo2, Wt1, bt1, Wt2, bt2):
    grid = (_B // _TILE,)

    def full(a):
        return pl.BlockSpec(a.shape, lambda i: (0,) * a.ndim)

    Ws2 = Ws2.reshape(_H, _NC, _CW).transpose(0, 2, 1).reshape(_H, _F * _F)
    bs2 = bs2.reshape(_NC, _CW).T.reshape(_F * _F)

    bs1r = bs1.reshape(1, _H)
    bs2r = bs2.reshape(1, _F * _F)
    bo1r = bo1.reshape(1, _H)
    bo2r = bo2.reshape(1, 4)
    bt1r = bt1.reshape(1, _H)
    bt2r = bt2.reshape(1, _F)

    out_shapes = (
        jax.ShapeDtypeStruct((_B, _K), jnp.float32),
        jax.ShapeDtypeStruct((1, _K), jnp.float32),
        jax.ShapeDtypeStruct((1, 4), jnp.float32),
        jax.ShapeDtypeStruct((1, _K), jnp.float32),
        jax.ShapeDtypeStruct((1, 1), jnp.float32),
    )
    out_specs = (
        pl.BlockSpec((_TILE, _K), lambda i: (i, 0)),
        pl.BlockSpec((1, _K), lambda i: (0, 0)),
        pl.BlockSpec((1, 4), lambda i: (0, 0)),
        pl.BlockSpec((1, _K), lambda i: (0, 0)),
        pl.BlockSpec((1, 1), lambda i: (0, 0)),
    )
    in_specs = [
        pl.BlockSpec((_TILE, _F), lambda i: (i, 0)),
        full(Ws1), full(bs1r), full(Ws2), full(bs2r),
        full(Wo1), full(bo1r), full(Wo2), full(bo2r),
        full(Wt1), full(bt1r), full(Wt2), full(bt2r),
    ]

    ratio_t, pv, op, mag, ent = pl.pallas_call(
        _body,
        grid=grid,
        in_specs=in_specs,
        out_specs=out_specs,
        out_shape=out_shapes,
    )(x, Ws1, bs1r, Ws2, bs2r, Wo1, bo1r, Wo2, bo2r, Wt1, bt1r, Wt2, bt2r)

    inv_b = 1.0 / _B
    return (ratio_t,
            pv[0] * inv_b,
            op[0] * inv_b,
            mag[0] * inv_b,
            (ent[0, 0] * inv_b).astype(jnp.float32))


# speedup vs baseline: 2.2123x; 1.0846x over previous
import jax
import jax.numpy as jnp
from jax.experimental import pallas as pl

_B = 4096
_F = 128
_H = 64
_K = 15
_EPS = 1e-08
_TILE = 128
_CW = 32
_NC = (_F * _F) // _CW
_CG = _NC // 128


def _body(x_ref, Ws1_ref, bs1_ref, Ws2_ref, bs2_ref, Wo1_ref, bo1_ref,
          Wo2_ref, bo2_ref, Wt1_ref, bt1_ref, Wt2_ref, bt2_ref,
          ratio_ref, pv_ref, op_ref, mag_ref, ent_ref):
    xb = x_ref[...]

    hs = jnp.maximum(jnp.dot(xb, Ws1_ref[...]) + bs1_ref[...], 0.0)
    logits = jnp.dot(hs, Ws2_ref[...]) + bs2_ref[...]
    sel = jax.nn.sigmoid(logits)

    ent_part = jnp.sum(-sel * jnp.log(sel + 1e-08)).reshape(1, 1)

    T = sel.shape[0]
    s4 = sel.reshape(T, _CW, _CG, 128)
    cmax = jnp.max(sel.reshape(T, _CW, _NC), axis=1)
    cid = jax.lax.broadcasted_iota(jnp.int32, cmax.shape, 1)
    chunk_ids = []
    for _ in range(_K):
        m = jnp.max(cmax, axis=1, keepdims=True)
        sel_cid = jnp.min(jnp.where(cmax == m, cid, _NC),
                          axis=1, keepdims=True)
        cmax = jnp.where(cid == sel_cid, -1.0, cmax)
        chunk_ids.append(sel_cid)
    csel = jnp.concatenate(chunk_ids, axis=1)
    cl = jnp.broadcast_to((csel % 128).reshape(T, 1, 1, _K),
                          (T, _CW, _CG, _K))
    gath = jnp.take_along_axis(s4, cl, axis=3)
    cg_s = (csel // 128).reshape(T, 1, 1, _K)
    g_iota = jax.lax.broadcasted_iota(jnp.int32, gath.shape, 2)
    cand = jnp.max(jnp.where(g_iota == cg_s, gath, -1.0),
                   axis=2).reshape(T, _K * _CW)
    q_iota = jax.lax.broadcasted_iota(jnp.int32, cand.shape, 1)
    s_pat = q_iota % _K
    w_pat = q_iota // _K
    oidx = jnp.take_along_axis(csel, s_pat, axis=1) * _CW + w_pat
    vals, idxs = [], []
    for _ in range(_K):
        m = jnp.max(cand, axis=1, keepdims=True)
        o = jnp.min(jnp.where(cand == m, oidx, _F * _F),
                    axis=1, keepdims=True)
        cand = jnp.where(oidx == o, -1.0, cand)
        vals.append(m)
        idxs.append(o)
    top_vals = jnp.concatenate(vals, axis=1)

    ht = jnp.maximum(jnp.dot(xb, Wt1_ref[...]) + bt1_ref[...], 0.0)
    tr = jnp.dot(ht, Wt2_ref[...]) + bt2_ref[...]

    ho = jnp.maximum(jnp.dot(xb, Wo1_ref[...]) + bo1_ref[...], 0.0)
    ol = jnp.dot(ho, Wo2_ref[...]) + bo2_ref[...]
    ol = ol - jax.lax.stop_gradient(jnp.max(ol, axis=1, keepdims=True))
    eol = jnp.exp(ol)
    opw = eol / jnp.sum(eol, axis=1, keepdims=True)

    lane = jax.lax.broadcasted_iota(jnp.int32, tr.shape, 1)
    w0 = opw[:, 0:1]
    w1 = opw[:, 1:2]
    w2 = opw[:, 2:3]
    w3 = opw[:, 3:4]
    cols = []
    for k in range(_K):
        i_k = idxs[k] // _F
        j_k = idxs[k] % _F
        fi = jnp.sum(jnp.where(lane == i_k, tr, 0.0), axis=1, keepdims=True)
        fj = jnp.sum(jnp.where(lane == j_k, tr, 0.0), axis=1, keepdims=True)
        abs_fj = jnp.abs(fj) + _EPS
        ratio = fi / abs_fj
        log_ratio = jnp.log(jnp.abs(fi) + _EPS) - jnp.log(abs_fj)
        combined = (ratio * w0 + log_ratio * w1 + (fi - fj) * w2
                    + (fi * fj) * w3)
        cols.append(combined)
    combined = jnp.concatenate(cols, axis=1)

    ratio_ref[...] = combined

    pv_part = jnp.sum(top_vals, axis=0, keepdims=True)
    mag_part = jnp.sum(jnp.abs(combined), axis=0, keepdims=True)
    op_part = jnp.sum(opw, axis=0, keepdims=True)

    @pl.when(pl.program_id(0) == 0)
    def _init():
        pv_ref[...] = pv_part
        op_ref[...] = op_part
        mag_ref[...] = mag_part
        ent_ref[...] = ent_part

    @pl.when(pl.program_id(0) != 0)
    def _acc():
        pv_ref[...] += pv_part
        op_ref[...] += op_part
        mag_ref[...] += mag_part
        ent_ref[...] += ent_part


def kernel(x, Ws1, bs1, Ws2, bs2, Wo1, bo1, Wo2, bo2, Wt1, bt1, Wt2, bt2):
    grid = (_B // _TILE,)

    def full(a):
        return pl.BlockSpec(a.shape, lambda i: (0,) * a.ndim)

    Ws2 = Ws2.reshape(_H, _NC, _CW).transpose(0, 2, 1).reshape(_H, _F * _F)
    bs2 = bs2.reshape(_NC, _CW).T.reshape(_F * _F)

    bs1r = bs1.reshape(1, _H)
    bs2r = bs2.reshape(1, _F * _F)
    bo1r = bo1.reshape(1, _H)
    bo2r = bo2.reshape(1, 4)
    bt1r = bt1.reshape(1, _H)
    bt2r = bt2.reshape(1, _F)

    out_shapes = (
        jax.ShapeDtypeStruct((_B, _K), jnp.float32),
        jax.ShapeDtypeStruct((1, _K), jnp.float32),
        jax.ShapeDtypeStruct((1, 4), jnp.float32),
        jax.ShapeDtypeStruct((1, _K), jnp.float32),
        jax.ShapeDtypeStruct((1, 1), jnp.float32),
    )
    out_specs = (
        pl.BlockSpec((_TILE, _K), lambda i: (i, 0)),
        pl.BlockSpec((1, _K), lambda i: (0, 0)),
        pl.BlockSpec((1, 4), lambda i: (0, 0)),
        pl.BlockSpec((1, _K), lambda i: (0, 0)),
        pl.BlockSpec((1, 1), lambda i: (0, 0)),
    )
    in_specs = [
        pl.BlockSpec((_TILE, _F), lambda i: (i, 0)),
        full(Ws1), full(bs1r), full(Ws2), full(bs2r),
        full(Wo1), full(bo1r), full(Wo2), full(bo2r),
        full(Wt1), full(bt1r), full(Wt2), full(bt2r),
    ]

    ratio_t, pv, op, mag, ent = pl.pallas_call(
        _body,
        grid=grid,
        in_specs=in_specs,
        out_specs=out_specs,
        out_shape=out_shapes,
    )(x, Ws1, bs1r, Ws2, bs2r, Wo1, bo1r, Wo2, bo2r, Wt1, bt1r, Wt2, bt2r)

    inv_b = 1.0 / _B
    return (ratio_t,
            pv[0] * inv_b,
            op[0] * inv_b,
            mag[0] * inv_b,
            (ent[0, 0] * inv_b).astype(jnp.float32))
